# baseline (speedup 1.0000x reference)
import functools

import jax
import jax.numpy as jnp
from jax import lax
from jax.experimental import pallas as pl
from jax.experimental.pallas import tpu as pltpu
from jax.experimental.pallas import tpu_sc as plsc

_LOG1P = (
    1.472065010887924e-06,
    0.9998476974962351,
    -0.49737321615793884,
    0.3157473167579205,
    -0.19035433673298097,
    0.08269123711134978,
    -0.017414077524237504,
)


def _log1p_poly(u):
    acc = jnp.full(u.shape, _LOG1P[-1], jnp.float32)
    for c in _LOG1P[-2::-1]:
        acc = acc * u + c
    return acc


def _neg_logit(lv, wv):
    t = -wv
    e = jnp.exp(-jnp.abs(wv))
    sp = jnp.maximum(t, 0.0) + _log1p_poly(e)
    return lv - sp


def _make_sc_kernel(N, D, NC, NS, C):
    NW = NC * NS
    npw = N // NW
    n_chunks = npw // C
    mesh = plsc.VectorSubcoreMesh(core_axis_name="c", subcore_axis_name="s")

    @functools.partial(
        pl.kernel,
        out_type=jax.ShapeDtypeStruct((N, 2, D), jnp.float32),
        mesh=mesh,
        scratch_types=[
            pltpu.VMEM((C,), jnp.int32),
            pltpu.VMEM((C, D), jnp.float32),
            pltpu.VMEM((C, D), jnp.float32),
            pltpu.VMEM((C, 2, D), jnp.float32),
            pltpu.SemaphoreType.DMA,
            pltpu.SemaphoreType.DMA,
        ],
        compiler_params=pltpu.CompilerParams(use_tc_tiling_on_sc=False),
    )
    def twin_gather(x_hbm, logits_hbm, weight_hbm, out_hbm,
                    idx_v, l_v, w_v, o_v, sem_l, sem_w):
        wid = lax.axis_index("s") * NC + lax.axis_index("c")
        base = wid * npw

        def chunk_body(ci, carry):
            off = base + ci * C
            pltpu.sync_copy(x_hbm.at[pl.ds(off, C)], idx_v)
            cp_l = pltpu.async_copy(logits_hbm.at[idx_v], l_v, sem_l)
            cp_w = pltpu.async_copy(weight_hbm.at[idx_v], w_v, sem_w)
            cp_l.wait()
            cp_w.wait()

            def row_body(i, carry2):
                for h in range(D // 16):
                    sl = pl.ds(h * 16, 16)
                    lv = l_v[i, sl]
                    wv = w_v[i, sl]
                    o_v[i, 0, sl] = lv
                    o_v[i, 1, sl] = _neg_logit(lv, wv)
                return carry2

            lax.fori_loop(0, C, row_body, 0)
            pltpu.sync_copy(o_v, out_hbm.at[pl.ds(off, C)])
            return carry

        lax.fori_loop(0, n_chunks, chunk_body, 0)

    return twin_gather


def kernel(x, logits, weight):
    B, L = x.shape
    V, D = logits.shape
    N = B * L
    info = plsc.get_sparse_core_info()
    NC, NS = info.num_cores, info.num_subcores
    xf = x.reshape(N).astype(jnp.int32)
    sc = _make_sc_kernel(N, D, NC, NS, C=512)
    out = sc(xf, logits, weight)
    return out.reshape(B, L, 2, D)


# speedup vs baseline: 1.3361x; 1.3361x over previous
import functools

import jax
import jax.numpy as jnp
from jax import lax
from jax.experimental import pallas as pl
from jax.experimental.pallas import tpu as pltpu
from jax.experimental.pallas import tpu_sc as plsc

_LOG1P = (
    1.472065010887924e-06,
    0.9998476974962351,
    -0.49737321615793884,
    0.3157473167579205,
    -0.19035433673298097,
    0.08269123711134978,
    -0.017414077524237504,
)


def _neg_logit(lv, wv):
    e = jnp.exp(jnp.minimum(wv, -wv))
    acc = jnp.full(lv.shape, _LOG1P[-1], jnp.float32)
    for c in _LOG1P[-2::-1]:
        acc = acc * e + c
    sp = jnp.maximum(-wv, 0.0) + acc
    return lv - sp


def _make_sc_kernel(N, D, NC, NS, C, NBUF):
    NW = NC * NS
    npw = N // NW
    n_chunks = npw // C
    mesh = plsc.VectorSubcoreMesh(core_axis_name="c", subcore_axis_name="s")

    @functools.partial(
        pl.kernel,
        out_type=jax.ShapeDtypeStruct((N, 2, D), jnp.float32),
        mesh=mesh,
        scratch_types=[
            pltpu.VMEM((NBUF, C), jnp.int32),
            pltpu.VMEM((NBUF, C, D), jnp.float32),
            pltpu.VMEM((NBUF, C, D), jnp.float32),
            pltpu.SemaphoreType.DMA((NBUF,)),
            pltpu.SemaphoreType.DMA((NBUF,)),
            pltpu.SemaphoreType.DMA((NBUF,)),
            pltpu.SemaphoreType.DMA((NBUF,)),
        ],
        compiler_params=pltpu.CompilerParams(use_tc_tiling_on_sc=False),
    )
    def twin_gather(x_hbm, logits_hbm, weight_hbm, out_hbm,
                    idx_v, l_v, w_v, sem_l, sem_w, sem_p, sem_n):
        wid = lax.axis_index("s") * NC + lax.axis_index("c")
        base = wid * npw

        def fire_gather(ci):
            b = ci % NBUF
            off = base + ci * C
            pltpu.sync_copy(x_hbm.at[pl.ds(off, C)], idx_v.at[b])
            gl = pltpu.async_copy(logits_hbm.at[idx_v.at[b]], l_v.at[b],
                                  sem_l.at[b])
            gw = pltpu.async_copy(weight_hbm.at[idx_v.at[b]], w_v.at[b],
                                  sem_w.at[b])
            return gl, gw

        gathers = {}
        outs = {}
        gathers[0] = fire_gather(0)
        gathers[1] = fire_gather(1)

        for ci in range(n_chunks):
            b = ci % NBUF
            off = base + ci * C
            gl, gw = gathers.pop(ci)
            gl.wait()
            gw.wait()

            lb = l_v.at[b]
            wb = w_v.at[b]

            @plsc.parallel_loop(0, C, unroll=4)
            def row_body(i):
                for h in range(D // 16):
                    sl = pl.ds(h * 16, 16)
                    wb[i, sl] = _neg_logit(lb[i, sl], wb[i, sl])

            cp_p = pltpu.async_copy(lb, out_hbm.at[pl.ds(off, C), 0],
                                    sem_p.at[b])
            cp_n = pltpu.async_copy(wb, out_hbm.at[pl.ds(off, C), 1],
                                    sem_n.at[b])
            outs[ci] = (cp_p, cp_n)

            nxt = ci + 2
            if nxt < n_chunks:
                prev = nxt - NBUF
                if prev >= 0:
                    op, on = outs.pop(prev)
                    op.wait()
                    on.wait()
                gathers[nxt] = fire_gather(nxt)

        for ci in sorted(outs):
            op, on = outs[ci]
            op.wait()
            on.wait()

    return twin_gather


def kernel(x, logits, weight):
    B, L = x.shape
    V, D = logits.shape
    N = B * L
    info = plsc.get_sparse_core_info()
    NC, NS = info.num_cores, info.num_subcores
    xf = x.reshape(N).astype(jnp.int32)
    sc = _make_sc_kernel(N, D, NC, NS, C=416, NBUF=4)
    out = sc(xf, logits, weight)
    return out.reshape(B, L, 2, D)
